# baseline (speedup 1.0000x reference)
import functools

import jax
import jax.numpy as jnp
from jax import lax
from jax.experimental import pallas as pl
from jax.experimental.pallas import tpu as pltpu
from jax.experimental.pallas import tpu_sc as plsc

N_NODES = 10000
D = 128
N_PAD = 10240
NC, NS = 2, 16
NW = NC * NS
EB = 128
KBLK = 4
ROWS_PER_TILE = N_PAD // NS


def _sc_degree(dst_w, zeros_1d, ones_eb, nblk2_by_core):
  @functools.partial(
      pl.kernel,
      out_type=jax.ShapeDtypeStruct((NC, N_PAD), jnp.float32),
      mesh=plsc.VectorSubcoreMesh(core_axis_name="c", subcore_axis_name="s"),
      scratch_types=[
          pltpu.VMEM((2, KBLK, EB), jnp.int32),
          pltpu.VMEM((EB,), jnp.float32),
          pltpu.VMEM_SHARED((N_PAD,), jnp.float32),
          pltpu.SemaphoreType.DMA,
      ],
  )
  def k(dst_hbm, z_hbm, ones_hbm, out_hbm, dst_v, ones_v, acc_sh, sem):
    c = lax.axis_index("c")
    s = lax.axis_index("s")
    wid = c * NS + s
    nblk2 = jnp.where(c == 0, nblk2_by_core[0], nblk2_by_core[1])
    pltpu.sync_copy(z_hbm, acc_sh.at[pl.ds(s * ROWS_PER_TILE, ROWS_PER_TILE)])
    pltpu.sync_copy(ones_hbm, ones_v)
    pltpu.async_copy(dst_hbm.at[wid].at[pl.ds(0, KBLK)], dst_v.at[0], sem)
    plsc.subcore_barrier()

    def blk(m, p):
      pltpu.make_async_copy(dst_hbm.at[wid].at[pl.ds(0, KBLK)], dst_v.at[p],
                            sem).wait()
      pltpu.async_copy(dst_hbm.at[wid].at[pl.ds((m + 1) * KBLK, KBLK)],
                       dst_v.at[1 - p], sem)
      for j in range(KBLK):
        pltpu.sync_copy(ones_v, acc_sh.at[dst_v.at[p].at[j]], add=True)

    def body(t, carry):
      blk(t * 2, 0)
      blk(t * 2 + 1, 1)
      return carry

    lax.fori_loop(0, nblk2, body, 0)
    pltpu.make_async_copy(dst_hbm.at[wid].at[pl.ds(0, KBLK)], dst_v.at[0],
                          sem).wait()
    plsc.subcore_barrier()
    pltpu.sync_copy(acc_sh.at[pl.ds(s * ROWS_PER_TILE, ROWS_PER_TILE)],
                    out_hbm.at[c].at[pl.ds(s * ROWS_PER_TILE, ROWS_PER_TILE)])

  return k(dst_w, zeros_1d, ones_eb)


def _sc_scatter(g, src_w, dst_w, zeros_2d, nblk2_by_core):
  @functools.partial(
      pl.kernel,
      out_type=jax.ShapeDtypeStruct((NC, N_PAD, D), jnp.float32),
      mesh=plsc.VectorSubcoreMesh(core_axis_name="c", subcore_axis_name="s"),
      scratch_types=[
          pltpu.VMEM((2, KBLK, EB), jnp.int32),
          pltpu.VMEM((2, KBLK, EB), jnp.int32),
          pltpu.VMEM((EB, D), jnp.float32),
          pltpu.VMEM_SHARED((N_PAD, D), jnp.float32),
          pltpu.SemaphoreType.DMA,
          pltpu.SemaphoreType.DMA,
      ],
  )
  def k(g_hbm, src_hbm, dst_hbm, z_hbm, out_hbm, src_v, dst_v, rows_v, acc_sh,
        sem_i, sem_g):
    c = lax.axis_index("c")
    s = lax.axis_index("s")
    wid = c * NS + s
    nblk2 = jnp.where(c == 0, nblk2_by_core[0], nblk2_by_core[1])
    pltpu.sync_copy(z_hbm, acc_sh.at[pl.ds(s * ROWS_PER_TILE, ROWS_PER_TILE)])
    pltpu.async_copy(src_hbm.at[wid].at[pl.ds(0, KBLK)], src_v.at[0], sem_i)
    pltpu.async_copy(dst_hbm.at[wid].at[pl.ds(0, KBLK)], dst_v.at[0], sem_i)
    plsc.subcore_barrier()

    def blk(m, p):
      pltpu.make_async_copy(src_hbm.at[wid].at[pl.ds(0, KBLK)], src_v.at[p],
                            sem_i).wait()
      pltpu.make_async_copy(dst_hbm.at[wid].at[pl.ds(0, KBLK)], dst_v.at[p],
                            sem_i).wait()
      pltpu.async_copy(src_hbm.at[wid].at[pl.ds((m + 1) * KBLK, KBLK)],
                       src_v.at[1 - p], sem_i)
      pltpu.async_copy(dst_hbm.at[wid].at[pl.ds((m + 1) * KBLK, KBLK)],
                       dst_v.at[1 - p], sem_i)
      for j in range(KBLK):
        pltpu.async_copy(g_hbm.at[src_v.at[p].at[j]], rows_v, sem_g).wait()
        pltpu.sync_copy(rows_v, acc_sh.at[dst_v.at[p].at[j]], add=True)

    def body(t, carry):
      blk(t * 2, 0)
      blk(t * 2 + 1, 1)
      return carry

    lax.fori_loop(0, nblk2, body, 0)
    pltpu.make_async_copy(src_hbm.at[wid].at[pl.ds(0, KBLK)], src_v.at[0],
                          sem_i).wait()
    pltpu.make_async_copy(dst_hbm.at[wid].at[pl.ds(0, KBLK)], dst_v.at[0],
                          sem_i).wait()
    plsc.subcore_barrier()
    pltpu.sync_copy(acc_sh.at[pl.ds(s * ROWS_PER_TILE, ROWS_PER_TILE)],
                    out_hbm.at[c].at[pl.ds(s * ROWS_PER_TILE, ROWS_PER_TILE)])

  return k(g, src_w, dst_w, zeros_2d)


_BLK = 1024
_GRID = N_PAD // _BLK


def _dis_body(p_ref, o_ref):
  o_ref[:] = lax.rsqrt(p_ref[0] + p_ref[1] + 1.0)


def _tc_dis(deg_parts):
  return pl.pallas_call(
      _dis_body,
      out_shape=jax.ShapeDtypeStruct((N_PAD // 128, 128), jnp.float32),
  )(deg_parts)


def _g0_body(x_ref, w_ref, dis_ref, o_ref):
  m = jnp.dot(x_ref[:], w_ref[:], preferred_element_type=jnp.float32)
  o_ref[:] = m * dis_ref[:]


def _tc_g0(x_pad, w0, dis_col):
  return pl.pallas_call(
      _g0_body,
      grid=(_GRID,),
      in_specs=[
          pl.BlockSpec((_BLK, D), lambda i: (i, 0)),
          pl.BlockSpec((D, D), lambda i: (0, 0)),
          pl.BlockSpec((_BLK, 1), lambda i: (i, 0)),
      ],
      out_specs=pl.BlockSpec((_BLK, D), lambda i: (i, 0)),
      out_shape=jax.ShapeDtypeStruct((N_PAD, D), jnp.float32),
  )(x_pad, w0, dis_col)


def _mid_body(s_ref, g_ref, dis_ref, b0_ref, gam_ref, bet_ref, w1_ref, o_ref):
  dis = dis_ref[:]
  t = (s_ref[0] + s_ref[1] + g_ref[:]) * dis + b0_ref[:]
  mu = jnp.mean(t, axis=1, keepdims=True)
  var = jnp.mean((t - mu) * (t - mu), axis=1, keepdims=True)
  h = (t - mu) * lax.rsqrt(var + 1e-5) * gam_ref[:] + bet_ref[:]
  h = jnp.maximum(h, 0.0)
  o_ref[:] = jnp.dot(h, w1_ref[:], preferred_element_type=jnp.float32) * dis


def _tc_mid(s0, g0, dis_col, b0, gamma, beta, w1):
  return pl.pallas_call(
      _mid_body,
      grid=(_GRID,),
      in_specs=[
          pl.BlockSpec((NC, _BLK, D), lambda i: (0, i, 0)),
          pl.BlockSpec((_BLK, D), lambda i: (i, 0)),
          pl.BlockSpec((_BLK, 1), lambda i: (i, 0)),
          pl.BlockSpec((1, D), lambda i: (0, 0)),
          pl.BlockSpec((1, D), lambda i: (0, 0)),
          pl.BlockSpec((1, D), lambda i: (0, 0)),
          pl.BlockSpec((D, D), lambda i: (0, 0)),
      ],
      out_specs=pl.BlockSpec((_BLK, D), lambda i: (i, 0)),
      out_shape=jax.ShapeDtypeStruct((N_PAD, D), jnp.float32),
  )(s0, g0, dis_col, b0, gamma, beta, w1)


def _fin_body(s_ref, g_ref, dis_ref, b1_ref, o_ref):
  o_ref[:] = (s_ref[0] + s_ref[1] + g_ref[:]) * dis_ref[:] + b1_ref[:]


def _tc_fin(s1, g1, dis_col, b1):
  return pl.pallas_call(
      _fin_body,
      grid=(_GRID,),
      in_specs=[
          pl.BlockSpec((NC, _BLK, D), lambda i: (0, i, 0)),
          pl.BlockSpec((_BLK, D), lambda i: (i, 0)),
          pl.BlockSpec((_BLK, 1), lambda i: (i, 0)),
          pl.BlockSpec((1, D), lambda i: (0, 0)),
      ],
      out_specs=pl.BlockSpec((_BLK, D), lambda i: (i, 0)),
      out_shape=jax.ShapeDtypeStruct((N_PAD, D), jnp.float32),
  )(s1, g1, dis_col, b1)


def _split_edges(idx, e, ch0, ch1):
  ch_alloc = max(ch0, ch1) + KBLK
  e0 = min(e, NS * ch0 * EB)
  p0 = jnp.full((NS * ch0 * EB - e0,), N_NODES, jnp.int32)
  w0 = jnp.concatenate([idx[:e0], p0]).reshape(NS, ch0, EB)
  w0 = jnp.pad(w0, ((0, 0), (0, ch_alloc - ch0), (0, 0)),
               constant_values=N_NODES)
  p1 = jnp.full((NS * ch1 * EB - (e - e0),), N_NODES, jnp.int32)
  w1 = jnp.concatenate([idx[e0:], p1]).reshape(NS, ch1, EB)
  w1 = jnp.pad(w1, ((0, 0), (0, ch_alloc - ch1), (0, 0)),
               constant_values=N_NODES)
  return jnp.concatenate([w0, w1], axis=0)


def kernel(x, edge_index, W0, b0, gamma, beta, W1, b1):
  n, d = x.shape
  e = edge_index.shape[1]
  def ceil_to(a, m):
    return -(-a // m) * m

  t_chunks = -(-e // (NS * EB))
  ch0 = min(ceil_to((36 * t_chunks) // 100, 2 * KBLK),
            ceil_to(t_chunks, 2 * KBLK))
  rem = max(e - NS * ch0 * EB, 0)
  ch1 = max(ceil_to(-(-rem // (NS * EB)), 2 * KBLK), 2 * KBLK)
  nblk2 = (ch0 // (2 * KBLK), ch1 // (2 * KBLK))

  src = edge_index[0].astype(jnp.int32)
  dst = edge_index[1].astype(jnp.int32)
  src_w = _split_edges(src, e, ch0, ch1)
  dst_w = _split_edges(dst, e, ch0, ch1)

  x_pad = jnp.zeros((N_PAD, d), x.dtype).at[:n].set(x)
  zeros_1d = jnp.zeros((ROWS_PER_TILE,), jnp.float32)
  zeros_2d = jnp.zeros((ROWS_PER_TILE, D), jnp.float32)
  ones_eb = jnp.ones((EB,), jnp.float32)
  b0r = b0.reshape(1, D)
  b1r = b1.reshape(1, D)
  gammar = gamma.reshape(1, D)
  betar = beta.reshape(1, D)

  deg_parts = _sc_degree(dst_w, zeros_1d, ones_eb, nblk2)
  dis2d = _tc_dis(deg_parts.reshape(NC, N_PAD // 128, 128))
  dis_col = dis2d.reshape(N_PAD, 1)

  g0 = _tc_g0(x_pad, W0, dis_col)
  s0 = _sc_scatter(g0, src_w, dst_w, zeros_2d, nblk2)
  g1 = _tc_mid(s0, g0, dis_col, b0r, gammar, betar, W1)
  s1 = _sc_scatter(g1, src_w, dst_w, zeros_2d, nblk2)
  out = _tc_fin(s1, g1, dis_col, b1r)
  return out[:n]


# speedup vs baseline: 1.1067x; 1.1067x over previous
import functools

import jax
import jax.numpy as jnp
from jax import lax
from jax.experimental import pallas as pl
from jax.experimental.pallas import tpu as pltpu
from jax.experimental.pallas import tpu_sc as plsc

N_NODES = 10000
D = 128
N_PAD = 10240
NC, NS = 2, 16
NW = NC * NS
EB = 128
KBLK = 4
ROWS_PER_TILE = N_PAD // NS


def _sc_degree(dst_w, zeros_1d, ones_eb, nblk2_by_core):
  @functools.partial(
      pl.kernel,
      out_type=jax.ShapeDtypeStruct((NC, N_PAD), jnp.float32),
      mesh=plsc.VectorSubcoreMesh(core_axis_name="c", subcore_axis_name="s"),
      scratch_types=[
          pltpu.VMEM((2, KBLK, EB), jnp.int32),
          pltpu.VMEM((EB,), jnp.float32),
          pltpu.VMEM_SHARED((N_PAD,), jnp.float32),
          pltpu.SemaphoreType.DMA,
      ],
  )
  def k(dst_hbm, z_hbm, ones_hbm, out_hbm, dst_v, ones_v, acc_sh, sem):
    c = lax.axis_index("c")
    s = lax.axis_index("s")
    wid = c * NS + s
    nblk2 = jnp.where(c == 0, nblk2_by_core[0], nblk2_by_core[1])
    pltpu.sync_copy(z_hbm, acc_sh.at[pl.ds(s * ROWS_PER_TILE, ROWS_PER_TILE)])
    pltpu.sync_copy(ones_hbm, ones_v)
    pltpu.async_copy(dst_hbm.at[wid].at[pl.ds(0, KBLK)], dst_v.at[0], sem)
    plsc.subcore_barrier()

    def blk(m, p):
      pltpu.make_async_copy(dst_hbm.at[wid].at[pl.ds(0, KBLK)], dst_v.at[p],
                            sem).wait()
      pltpu.async_copy(dst_hbm.at[wid].at[pl.ds((m + 1) * KBLK, KBLK)],
                       dst_v.at[1 - p], sem)
      for j in range(KBLK):
        pltpu.sync_copy(ones_v, acc_sh.at[dst_v.at[p].at[j]], add=True)

    def body(t, carry):
      blk(t * 2, 0)
      blk(t * 2 + 1, 1)
      return carry

    lax.fori_loop(0, nblk2, body, 0)
    pltpu.make_async_copy(dst_hbm.at[wid].at[pl.ds(0, KBLK)], dst_v.at[0],
                          sem).wait()
    plsc.subcore_barrier()
    pltpu.sync_copy(acc_sh.at[pl.ds(s * ROWS_PER_TILE, ROWS_PER_TILE)],
                    out_hbm.at[c].at[pl.ds(s * ROWS_PER_TILE, ROWS_PER_TILE)])

  return k(dst_w, zeros_1d, ones_eb)


def _sc_scatter(g, src_w, dst_w, zeros_2d, nblk2_by_core):
  @functools.partial(
      pl.kernel,
      out_type=jax.ShapeDtypeStruct((NC, N_PAD, D), jnp.float32),
      mesh=plsc.VectorSubcoreMesh(core_axis_name="c", subcore_axis_name="s"),
      scratch_types=[
          pltpu.VMEM((2, KBLK, EB), jnp.int32),
          pltpu.VMEM((2, KBLK, EB), jnp.int32),
          pltpu.VMEM((EB, D), jnp.float32),
          pltpu.VMEM_SHARED((N_PAD, D), jnp.float32),
          pltpu.SemaphoreType.DMA,
          pltpu.SemaphoreType.DMA,
      ],
  )
  def k(g_hbm, src_hbm, dst_hbm, z_hbm, out_hbm, src_v, dst_v, rows_v, acc_sh,
        sem_i, sem_g):
    c = lax.axis_index("c")
    s = lax.axis_index("s")
    wid = c * NS + s
    nblk2 = jnp.where(c == 0, nblk2_by_core[0], nblk2_by_core[1])
    pltpu.sync_copy(z_hbm, acc_sh.at[pl.ds(s * ROWS_PER_TILE, ROWS_PER_TILE)])
    pltpu.async_copy(src_hbm.at[wid].at[pl.ds(0, KBLK)], src_v.at[0], sem_i)
    pltpu.async_copy(dst_hbm.at[wid].at[pl.ds(0, KBLK)], dst_v.at[0], sem_i)
    plsc.subcore_barrier()

    def blk(m, p):
      pltpu.make_async_copy(src_hbm.at[wid].at[pl.ds(0, KBLK)], src_v.at[p],
                            sem_i).wait()
      pltpu.make_async_copy(dst_hbm.at[wid].at[pl.ds(0, KBLK)], dst_v.at[p],
                            sem_i).wait()
      pltpu.async_copy(src_hbm.at[wid].at[pl.ds((m + 1) * KBLK, KBLK)],
                       src_v.at[1 - p], sem_i)
      pltpu.async_copy(dst_hbm.at[wid].at[pl.ds((m + 1) * KBLK, KBLK)],
                       dst_v.at[1 - p], sem_i)
      for j in range(KBLK):
        pltpu.async_copy(g_hbm.at[src_v.at[p].at[j]], rows_v, sem_g).wait()
        pltpu.sync_copy(rows_v, acc_sh.at[dst_v.at[p].at[j]], add=True)

    def body(t, carry):
      blk(t * 2, 0)
      blk(t * 2 + 1, 1)
      return carry

    lax.fori_loop(0, nblk2, body, 0)
    pltpu.make_async_copy(src_hbm.at[wid].at[pl.ds(0, KBLK)], src_v.at[0],
                          sem_i).wait()
    pltpu.make_async_copy(dst_hbm.at[wid].at[pl.ds(0, KBLK)], dst_v.at[0],
                          sem_i).wait()
    plsc.subcore_barrier()
    pltpu.sync_copy(acc_sh.at[pl.ds(s * ROWS_PER_TILE, ROWS_PER_TILE)],
                    out_hbm.at[c].at[pl.ds(s * ROWS_PER_TILE, ROWS_PER_TILE)])

  return k(g, src_w, dst_w, zeros_2d)


_BLK = 1024
_GRID = N_PAD // _BLK


def _dis_body(p_ref, o_ref):
  o_ref[:] = lax.rsqrt(p_ref[0] + p_ref[1] + 1.0)


def _tc_dis(deg_parts):
  return pl.pallas_call(
      _dis_body,
      out_shape=jax.ShapeDtypeStruct((N_PAD // 128, 128), jnp.float32),
  )(deg_parts)


def _g0_body(x_ref, w_ref, dis_ref, o_ref):
  m = jnp.dot(x_ref[:], w_ref[:], preferred_element_type=jnp.float32)
  o_ref[:] = m * dis_ref[:]


def _tc_g0(x_pad, w0, dis_col):
  return pl.pallas_call(
      _g0_body,
      grid=(_GRID,),
      in_specs=[
          pl.BlockSpec((_BLK, D), lambda i: (i, 0)),
          pl.BlockSpec((D, D), lambda i: (0, 0)),
          pl.BlockSpec((_BLK, 1), lambda i: (i, 0)),
      ],
      out_specs=pl.BlockSpec((_BLK, D), lambda i: (i, 0)),
      out_shape=jax.ShapeDtypeStruct((N_PAD, D), jnp.float32),
  )(x_pad, w0, dis_col)


def _mid_body(s_ref, g_ref, dis_ref, b0_ref, gam_ref, bet_ref, w1_ref, o_ref):
  dis = dis_ref[:]
  t = (s_ref[0] + s_ref[1] + g_ref[:]) * dis + b0_ref[:]
  mu = jnp.mean(t, axis=1, keepdims=True)
  var = jnp.mean((t - mu) * (t - mu), axis=1, keepdims=True)
  h = (t - mu) * lax.rsqrt(var + 1e-5) * gam_ref[:] + bet_ref[:]
  h = jnp.maximum(h, 0.0)
  o_ref[:] = jnp.dot(h, w1_ref[:], preferred_element_type=jnp.float32) * dis


def _tc_mid(s0, g0, dis_col, b0, gamma, beta, w1):
  return pl.pallas_call(
      _mid_body,
      grid=(_GRID,),
      in_specs=[
          pl.BlockSpec((NC, _BLK, D), lambda i: (0, i, 0)),
          pl.BlockSpec((_BLK, D), lambda i: (i, 0)),
          pl.BlockSpec((_BLK, 1), lambda i: (i, 0)),
          pl.BlockSpec((1, D), lambda i: (0, 0)),
          pl.BlockSpec((1, D), lambda i: (0, 0)),
          pl.BlockSpec((1, D), lambda i: (0, 0)),
          pl.BlockSpec((D, D), lambda i: (0, 0)),
      ],
      out_specs=pl.BlockSpec((_BLK, D), lambda i: (i, 0)),
      out_shape=jax.ShapeDtypeStruct((N_PAD, D), jnp.float32),
  )(s0, g0, dis_col, b0, gamma, beta, w1)


def _fin_body(s_ref, g_ref, dis_ref, b1_ref, o_ref):
  o_ref[:] = (s_ref[0] + s_ref[1] + g_ref[:]) * dis_ref[:] + b1_ref[:]


def _tc_fin(s1, g1, dis_col, b1):
  return pl.pallas_call(
      _fin_body,
      grid=(_GRID,),
      in_specs=[
          pl.BlockSpec((NC, _BLK, D), lambda i: (0, i, 0)),
          pl.BlockSpec((_BLK, D), lambda i: (i, 0)),
          pl.BlockSpec((_BLK, 1), lambda i: (i, 0)),
          pl.BlockSpec((1, D), lambda i: (0, 0)),
      ],
      out_specs=pl.BlockSpec((_BLK, D), lambda i: (i, 0)),
      out_shape=jax.ShapeDtypeStruct((N_PAD, D), jnp.float32),
  )(s1, g1, dis_col, b1)


def _split_edges(idx, e, ch0, ch1):
  ch_alloc = max(ch0, ch1) + KBLK
  e0 = min(e, NS * ch0 * EB)
  p0 = jnp.full((NS * ch0 * EB - e0,), N_NODES, jnp.int32)
  w0 = jnp.concatenate([idx[:e0], p0]).reshape(NS, ch0, EB)
  w0 = jnp.pad(w0, ((0, 0), (0, ch_alloc - ch0), (0, 0)),
               constant_values=N_NODES)
  p1 = jnp.full((NS * ch1 * EB - (e - e0),), N_NODES, jnp.int32)
  w1 = jnp.concatenate([idx[e0:], p1]).reshape(NS, ch1, EB)
  w1 = jnp.pad(w1, ((0, 0), (0, ch_alloc - ch1), (0, 0)),
               constant_values=N_NODES)
  return jnp.concatenate([w0, w1], axis=0)


def kernel(x, edge_index, W0, b0, gamma, beta, W1, b1):
  n, d = x.shape
  e = edge_index.shape[1]
  def ceil_to(a, m):
    return -(-a // m) * m

  t_chunks = -(-e // (NS * EB))
  ch0 = min(ceil_to((65 * t_chunks) // 100, 2 * KBLK),
            ceil_to(t_chunks, 2 * KBLK))
  rem = max(e - NS * ch0 * EB, 0)
  ch1 = max(ceil_to(-(-rem // (NS * EB)), 2 * KBLK), 2 * KBLK)
  nblk2 = (ch0 // (2 * KBLK), ch1 // (2 * KBLK))

  src = edge_index[0].astype(jnp.int32)
  dst = edge_index[1].astype(jnp.int32)
  src_w = _split_edges(src, e, ch0, ch1)
  dst_w = _split_edges(dst, e, ch0, ch1)

  x_pad = jnp.zeros((N_PAD, d), x.dtype).at[:n].set(x)
  zeros_1d = jnp.zeros((ROWS_PER_TILE,), jnp.float32)
  zeros_2d = jnp.zeros((ROWS_PER_TILE, D), jnp.float32)
  ones_eb = jnp.ones((EB,), jnp.float32)
  b0r = b0.reshape(1, D)
  b1r = b1.reshape(1, D)
  gammar = gamma.reshape(1, D)
  betar = beta.reshape(1, D)

  deg_parts = _sc_degree(dst_w, zeros_1d, ones_eb, nblk2)
  dis2d = _tc_dis(deg_parts.reshape(NC, N_PAD // 128, 128))
  dis_col = dis2d.reshape(N_PAD, 1)

  g0 = _tc_g0(x_pad, W0, dis_col)
  s0 = _sc_scatter(g0, src_w, dst_w, zeros_2d, nblk2)
  g1 = _tc_mid(s0, g0, dis_col, b0r, gammar, betar, W1)
  s1 = _sc_scatter(g1, src_w, dst_w, zeros_2d, nblk2)
  out = _tc_fin(s1, g1, dis_col, b1r)
  return out[:n]


# speedup vs baseline: 1.1083x; 1.0014x over previous
import functools

import jax
import jax.numpy as jnp
from jax import lax
from jax.experimental import pallas as pl
from jax.experimental.pallas import tpu as pltpu
from jax.experimental.pallas import tpu_sc as plsc

N_NODES = 10000
D = 128
N_PAD = 10240
NC, NS = 2, 16
NW = NC * NS
EB = 128
KBLK = 4
ROWS_PER_TILE = N_PAD // NS


def _sc_degree(dst_w, zeros_1d, ones_eb, nblk2_by_core):
  @functools.partial(
      pl.kernel,
      out_type=jax.ShapeDtypeStruct((NC, N_PAD), jnp.float32),
      mesh=plsc.VectorSubcoreMesh(core_axis_name="c", subcore_axis_name="s"),
      scratch_types=[
          pltpu.VMEM((2, KBLK, EB), jnp.int32),
          pltpu.VMEM((EB,), jnp.float32),
          pltpu.VMEM_SHARED((N_PAD,), jnp.float32),
          pltpu.SemaphoreType.DMA,
      ],
  )
  def k(dst_hbm, z_hbm, ones_hbm, out_hbm, dst_v, ones_v, acc_sh, sem):
    c = lax.axis_index("c")
    s = lax.axis_index("s")
    wid = c * NS + s
    nblk2 = jnp.where(c == 0, nblk2_by_core[0], nblk2_by_core[1])
    pltpu.sync_copy(z_hbm, acc_sh.at[pl.ds(s * ROWS_PER_TILE, ROWS_PER_TILE)])
    pltpu.sync_copy(ones_hbm, ones_v)
    pltpu.async_copy(dst_hbm.at[wid].at[pl.ds(0, KBLK)], dst_v.at[0], sem)
    plsc.subcore_barrier()

    def blk(m, p):
      pltpu.make_async_copy(dst_hbm.at[wid].at[pl.ds(0, KBLK)], dst_v.at[p],
                            sem).wait()
      pltpu.async_copy(dst_hbm.at[wid].at[pl.ds((m + 1) * KBLK, KBLK)],
                       dst_v.at[1 - p], sem)
      for j in range(KBLK):
        pltpu.sync_copy(ones_v, acc_sh.at[dst_v.at[p].at[j]], add=True)

    def body(t, carry):
      blk(t * 2, 0)
      blk(t * 2 + 1, 1)
      return carry

    lax.fori_loop(0, nblk2, body, 0)
    pltpu.make_async_copy(dst_hbm.at[wid].at[pl.ds(0, KBLK)], dst_v.at[0],
                          sem).wait()
    plsc.subcore_barrier()
    pltpu.sync_copy(acc_sh.at[pl.ds(s * ROWS_PER_TILE, ROWS_PER_TILE)],
                    out_hbm.at[c].at[pl.ds(s * ROWS_PER_TILE, ROWS_PER_TILE)])

  return k(dst_w, zeros_1d, ones_eb)


def _sc_scatter(g, src_w, dst_w, zeros_2d, nblk2_by_core):
  @functools.partial(
      pl.kernel,
      out_type=jax.ShapeDtypeStruct((NC, N_PAD, D), jnp.float32),
      mesh=plsc.VectorSubcoreMesh(core_axis_name="c", subcore_axis_name="s"),
      scratch_types=[
          pltpu.VMEM((2, KBLK, EB), jnp.int32),
          pltpu.VMEM((2, KBLK, EB), jnp.int32),
          pltpu.VMEM((EB, D), jnp.float32),
          pltpu.VMEM_SHARED((N_PAD, D), jnp.float32),
          pltpu.SemaphoreType.DMA,
          pltpu.SemaphoreType.DMA,
      ],
  )
  def k(g_hbm, src_hbm, dst_hbm, z_hbm, out_hbm, src_v, dst_v, rows_v, acc_sh,
        sem_i, sem_g):
    c = lax.axis_index("c")
    s = lax.axis_index("s")
    wid = c * NS + s
    nblk2 = jnp.where(c == 0, nblk2_by_core[0], nblk2_by_core[1])
    with jax.named_scope("zfill"):
      pltpu.sync_copy(z_hbm, acc_sh.at[pl.ds(s * ROWS_PER_TILE, ROWS_PER_TILE)])
      pltpu.async_copy(src_hbm.at[wid].at[pl.ds(0, KBLK)], src_v.at[0], sem_i)
      pltpu.async_copy(dst_hbm.at[wid].at[pl.ds(0, KBLK)], dst_v.at[0], sem_i)
      plsc.subcore_barrier()

    def blk(m, p):
      pltpu.make_async_copy(src_hbm.at[wid].at[pl.ds(0, KBLK)], src_v.at[p],
                            sem_i).wait()
      pltpu.make_async_copy(dst_hbm.at[wid].at[pl.ds(0, KBLK)], dst_v.at[p],
                            sem_i).wait()
      pltpu.async_copy(src_hbm.at[wid].at[pl.ds((m + 1) * KBLK, KBLK)],
                       src_v.at[1 - p], sem_i)
      pltpu.async_copy(dst_hbm.at[wid].at[pl.ds((m + 1) * KBLK, KBLK)],
                       dst_v.at[1 - p], sem_i)
      for j in range(KBLK):
        pltpu.async_copy(g_hbm.at[src_v.at[p].at[j]], rows_v, sem_g).wait()
        pltpu.sync_copy(rows_v, acc_sh.at[dst_v.at[p].at[j]], add=True)

    def body(t, carry):
      blk(t * 2, 0)
      blk(t * 2 + 1, 1)
      return carry

    with jax.named_scope("eloop"):
      lax.fori_loop(0, nblk2, body, 0)
      pltpu.make_async_copy(src_hbm.at[wid].at[pl.ds(0, KBLK)], src_v.at[0],
                            sem_i).wait()
      pltpu.make_async_copy(dst_hbm.at[wid].at[pl.ds(0, KBLK)], dst_v.at[0],
                            sem_i).wait()
    with jax.named_scope("obar"):
      plsc.subcore_barrier()
    with jax.named_scope("oout"):
      pltpu.sync_copy(acc_sh.at[pl.ds(s * ROWS_PER_TILE, ROWS_PER_TILE)],
                      out_hbm.at[c].at[pl.ds(s * ROWS_PER_TILE, ROWS_PER_TILE)])

  return k(g, src_w, dst_w, zeros_2d)


_BLK = 1024
_GRID = N_PAD // _BLK


def _dis_body(p_ref, o_ref):
  o_ref[:] = lax.rsqrt(p_ref[0] + p_ref[1] + 1.0)


def _tc_dis(deg_parts):
  return pl.pallas_call(
      _dis_body,
      out_shape=jax.ShapeDtypeStruct((N_PAD // 128, 128), jnp.float32),
  )(deg_parts)


def _g0_body(x_ref, w_ref, dis_ref, o_ref):
  m = jnp.dot(x_ref[:], w_ref[:], preferred_element_type=jnp.float32)
  o_ref[:] = m * dis_ref[:]


def _tc_g0(x_pad, w0, dis_col):
  return pl.pallas_call(
      _g0_body,
      grid=(_GRID,),
      in_specs=[
          pl.BlockSpec((_BLK, D), lambda i: (i, 0)),
          pl.BlockSpec((D, D), lambda i: (0, 0)),
          pl.BlockSpec((_BLK, 1), lambda i: (i, 0)),
      ],
      out_specs=pl.BlockSpec((_BLK, D), lambda i: (i, 0)),
      out_shape=jax.ShapeDtypeStruct((N_PAD, D), jnp.float32),
  )(x_pad, w0, dis_col)


def _mid_body(s_ref, g_ref, dis_ref, b0_ref, gam_ref, bet_ref, w1_ref, o_ref):
  dis = dis_ref[:]
  t = (s_ref[0] + s_ref[1] + g_ref[:]) * dis + b0_ref[:]
  mu = jnp.mean(t, axis=1, keepdims=True)
  var = jnp.mean((t - mu) * (t - mu), axis=1, keepdims=True)
  h = (t - mu) * lax.rsqrt(var + 1e-5) * gam_ref[:] + bet_ref[:]
  h = jnp.maximum(h, 0.0)
  o_ref[:] = jnp.dot(h, w1_ref[:], preferred_element_type=jnp.float32) * dis


def _tc_mid(s0, g0, dis_col, b0, gamma, beta, w1):
  return pl.pallas_call(
      _mid_body,
      grid=(_GRID,),
      in_specs=[
          pl.BlockSpec((NC, _BLK, D), lambda i: (0, i, 0)),
          pl.BlockSpec((_BLK, D), lambda i: (i, 0)),
          pl.BlockSpec((_BLK, 1), lambda i: (i, 0)),
          pl.BlockSpec((1, D), lambda i: (0, 0)),
          pl.BlockSpec((1, D), lambda i: (0, 0)),
          pl.BlockSpec((1, D), lambda i: (0, 0)),
          pl.BlockSpec((D, D), lambda i: (0, 0)),
      ],
      out_specs=pl.BlockSpec((_BLK, D), lambda i: (i, 0)),
      out_shape=jax.ShapeDtypeStruct((N_PAD, D), jnp.float32),
  )(s0, g0, dis_col, b0, gamma, beta, w1)


def _fin_body(s_ref, g_ref, dis_ref, b1_ref, o_ref):
  o_ref[:] = (s_ref[0] + s_ref[1] + g_ref[:]) * dis_ref[:] + b1_ref[:]


def _tc_fin(s1, g1, dis_col, b1):
  return pl.pallas_call(
      _fin_body,
      grid=(_GRID,),
      in_specs=[
          pl.BlockSpec((NC, _BLK, D), lambda i: (0, i, 0)),
          pl.BlockSpec((_BLK, D), lambda i: (i, 0)),
          pl.BlockSpec((_BLK, 1), lambda i: (i, 0)),
          pl.BlockSpec((1, D), lambda i: (0, 0)),
      ],
      out_specs=pl.BlockSpec((_BLK, D), lambda i: (i, 0)),
      out_shape=jax.ShapeDtypeStruct((N_PAD, D), jnp.float32),
  )(s1, g1, dis_col, b1)


def _split_edges(idx, e, ch0, ch1):
  ch_alloc = max(ch0, ch1) + KBLK
  e0 = min(e, NS * ch0 * EB)
  p0 = jnp.full((NS * ch0 * EB - e0,), N_NODES, jnp.int32)
  w0 = jnp.concatenate([idx[:e0], p0]).reshape(NS, ch0, EB)
  w0 = jnp.pad(w0, ((0, 0), (0, ch_alloc - ch0), (0, 0)),
               constant_values=N_NODES)
  p1 = jnp.full((NS * ch1 * EB - (e - e0),), N_NODES, jnp.int32)
  w1 = jnp.concatenate([idx[e0:], p1]).reshape(NS, ch1, EB)
  w1 = jnp.pad(w1, ((0, 0), (0, ch_alloc - ch1), (0, 0)),
               constant_values=N_NODES)
  return jnp.concatenate([w0, w1], axis=0)


def kernel(x, edge_index, W0, b0, gamma, beta, W1, b1):
  n, d = x.shape
  e = edge_index.shape[1]
  def ceil_to(a, m):
    return -(-a // m) * m

  t_chunks = -(-e // (NS * EB))
  ch0 = min(ceil_to((65 * t_chunks) // 100, 2 * KBLK),
            ceil_to(t_chunks, 2 * KBLK))
  rem = max(e - NS * ch0 * EB, 0)
  ch1 = max(ceil_to(-(-rem // (NS * EB)), 2 * KBLK), 2 * KBLK)
  nblk2 = (ch0 // (2 * KBLK), ch1 // (2 * KBLK))

  src = edge_index[0].astype(jnp.int32)
  dst = edge_index[1].astype(jnp.int32)
  src_w = _split_edges(src, e, ch0, ch1)
  dst_w = _split_edges(dst, e, ch0, ch1)

  x_pad = jnp.zeros((N_PAD, d), x.dtype).at[:n].set(x)
  zeros_1d = jnp.zeros((ROWS_PER_TILE,), jnp.float32)
  zeros_2d = jnp.zeros((ROWS_PER_TILE, D), jnp.float32)
  ones_eb = jnp.ones((EB,), jnp.float32)
  b0r = b0.reshape(1, D)
  b1r = b1.reshape(1, D)
  gammar = gamma.reshape(1, D)
  betar = beta.reshape(1, D)

  deg_parts = _sc_degree(dst_w, zeros_1d, ones_eb, nblk2)
  dis2d = _tc_dis(deg_parts.reshape(NC, N_PAD // 128, 128))
  dis_col = dis2d.reshape(N_PAD, 1)

  g0 = _tc_g0(x_pad, W0, dis_col)
  s0 = _sc_scatter(g0, src_w, dst_w, zeros_2d, nblk2)
  g1 = _tc_mid(s0, g0, dis_col, b0r, gammar, betar, W1)
  s1 = _sc_scatter(g1, src_w, dst_w, zeros_2d, nblk2)
  out = _tc_fin(s1, g1, dis_col, b1r)
  return out[:n]


# speedup vs baseline: 2.4930x; 2.2494x over previous
import functools

import jax
import jax.numpy as jnp
from jax import lax
from jax.experimental import pallas as pl
from jax.experimental.pallas import tpu as pltpu
from jax.experimental.pallas import tpu_sc as plsc

N_NODES = 10000
D = 128
N_PAD = 10240
NC, NS = 2, 16
NW = NC * NS
EB = 128
KBLK = 4
ROWS_PER_TILE = N_PAD // NS


def _sc_degree(dst_w, zeros_1d, ones_eb, nblk2_by_core):
  @functools.partial(
      pl.kernel,
      out_type=jax.ShapeDtypeStruct((NC, N_PAD), jnp.float32),
      mesh=plsc.VectorSubcoreMesh(core_axis_name="c", subcore_axis_name="s"),
      scratch_types=[
          pltpu.VMEM((2, KBLK, EB), jnp.int32),
          pltpu.VMEM((EB,), jnp.float32),
          pltpu.VMEM_SHARED((N_PAD,), jnp.float32),
          pltpu.SemaphoreType.DMA,
      ],
  )
  def k(dst_hbm, z_hbm, ones_hbm, out_hbm, dst_v, ones_v, acc_sh, sem):
    c = lax.axis_index("c")
    s = lax.axis_index("s")
    wid = c * NS + s
    nblk2 = jnp.where(c == 0, nblk2_by_core[0], nblk2_by_core[1])
    pltpu.sync_copy(z_hbm, acc_sh.at[pl.ds(s * ROWS_PER_TILE, ROWS_PER_TILE)])
    pltpu.sync_copy(ones_hbm, ones_v)
    pltpu.async_copy(dst_hbm.at[wid].at[pl.ds(0, KBLK)], dst_v.at[0], sem)
    plsc.subcore_barrier()

    def blk(m, p):
      pltpu.make_async_copy(dst_hbm.at[wid].at[pl.ds(0, KBLK)], dst_v.at[p],
                            sem).wait()
      pltpu.async_copy(dst_hbm.at[wid].at[pl.ds((m + 1) * KBLK, KBLK)],
                       dst_v.at[1 - p], sem)
      for j in range(KBLK):
        pltpu.sync_copy(ones_v, acc_sh.at[dst_v.at[p].at[j]], add=True)

    def body(t, carry):
      blk(t * 2, 0)
      blk(t * 2 + 1, 1)
      return carry

    lax.fori_loop(0, nblk2, body, 0)
    pltpu.make_async_copy(dst_hbm.at[wid].at[pl.ds(0, KBLK)], dst_v.at[0],
                          sem).wait()
    plsc.subcore_barrier()
    pltpu.sync_copy(acc_sh.at[pl.ds(s * ROWS_PER_TILE, ROWS_PER_TILE)],
                    out_hbm.at[c].at[pl.ds(s * ROWS_PER_TILE, ROWS_PER_TILE)])

  return k(dst_w, zeros_1d, ones_eb)


def _sc_scatter(g, src_w, dst_w, zeros_2d, nblk2_by_core):
  @functools.partial(
      pl.kernel,
      out_type=jax.ShapeDtypeStruct((NC, N_PAD, D), jnp.float32),
      mesh=plsc.VectorSubcoreMesh(core_axis_name="c", subcore_axis_name="s"),
      scratch_types=[
          pltpu.VMEM((2, KBLK, EB), jnp.int32),
          pltpu.VMEM((2, KBLK, EB), jnp.int32),
          pltpu.VMEM((EB, D), jnp.float32),
          pltpu.VMEM_SHARED((N_PAD, D), jnp.float32),
          pltpu.SemaphoreType.DMA,
          pltpu.SemaphoreType.DMA,
      ],
  )
  def k(g_hbm, src_hbm, dst_hbm, z_hbm, out_hbm, src_v, dst_v, rows_v, acc_sh,
        sem_i, sem_g):
    c = lax.axis_index("c")
    s = lax.axis_index("s")
    wid = c * NS + s
    nblk2 = jnp.where(c == 0, nblk2_by_core[0], nblk2_by_core[1])
    with jax.named_scope("zfill"):
      pltpu.sync_copy(z_hbm, acc_sh.at[pl.ds(s * ROWS_PER_TILE, ROWS_PER_TILE)])
      pltpu.async_copy(src_hbm.at[wid].at[pl.ds(0, KBLK)], src_v.at[0], sem_i)
      pltpu.async_copy(dst_hbm.at[wid].at[pl.ds(0, KBLK)], dst_v.at[0], sem_i)
      plsc.subcore_barrier()

    def blk(m, p):
      pltpu.make_async_copy(src_hbm.at[wid].at[pl.ds(0, KBLK)], src_v.at[p],
                            sem_i).wait()
      pltpu.make_async_copy(dst_hbm.at[wid].at[pl.ds(0, KBLK)], dst_v.at[p],
                            sem_i).wait()
      pltpu.async_copy(src_hbm.at[wid].at[pl.ds((m + 1) * KBLK, KBLK)],
                       src_v.at[1 - p], sem_i)
      pltpu.async_copy(dst_hbm.at[wid].at[pl.ds((m + 1) * KBLK, KBLK)],
                       dst_v.at[1 - p], sem_i)
      for j in range(KBLK):
        pltpu.async_copy(g_hbm.at[src_v.at[p].at[j]], rows_v, sem_g).wait()
        pltpu.sync_copy(rows_v, acc_sh.at[dst_v.at[p].at[j]], add=True)

    def body(t, carry):
      blk(t * 2, 0)
      blk(t * 2 + 1, 1)
      return carry

    with jax.named_scope("eloop"):
      lax.fori_loop(0, nblk2, body, 0)
      pltpu.make_async_copy(src_hbm.at[wid].at[pl.ds(0, KBLK)], src_v.at[0],
                            sem_i).wait()
      pltpu.make_async_copy(dst_hbm.at[wid].at[pl.ds(0, KBLK)], dst_v.at[0],
                            sem_i).wait()
    with jax.named_scope("obar"):
      plsc.subcore_barrier()
    with jax.named_scope("oout"):
      pltpu.sync_copy(acc_sh.at[pl.ds(s * ROWS_PER_TILE, ROWS_PER_TILE)],
                      out_hbm.at[c].at[pl.ds(s * ROWS_PER_TILE, ROWS_PER_TILE)])

  return k(g, src_w, dst_w, zeros_2d)


_BLK = 1024
_GRID = N_PAD // _BLK


def _dis_body(p_ref, o_ref):
  o_ref[:] = lax.rsqrt(p_ref[0] + p_ref[1] + 1.0)


def _tc_dis(deg_parts):
  return pl.pallas_call(
      _dis_body,
      out_shape=jax.ShapeDtypeStruct((N_PAD // 128, 128), jnp.float32),
  )(deg_parts)


def _g0_body(x_ref, w_ref, dis_ref, o_ref):
  m = jnp.dot(x_ref[:], w_ref[:], preferred_element_type=jnp.float32)
  o_ref[:] = m * dis_ref[:]


def _tc_g0(x_pad, w0, dis_col):
  return pl.pallas_call(
      _g0_body,
      grid=(_GRID,),
      in_specs=[
          pl.BlockSpec((_BLK, D), lambda i: (i, 0)),
          pl.BlockSpec((D, D), lambda i: (0, 0)),
          pl.BlockSpec((_BLK, 1), lambda i: (i, 0)),
      ],
      out_specs=pl.BlockSpec((_BLK, D), lambda i: (i, 0)),
      out_shape=jax.ShapeDtypeStruct((N_PAD, D), jnp.float32),
  )(x_pad, w0, dis_col)


def _mid_body(s_ref, g_ref, dis_ref, b0_ref, gam_ref, bet_ref, w1_ref, o_ref):
  dis = dis_ref[:]
  t = (s_ref[0] + s_ref[1] + g_ref[:]) * dis + b0_ref[:]
  mu = jnp.mean(t, axis=1, keepdims=True)
  var = jnp.mean((t - mu) * (t - mu), axis=1, keepdims=True)
  h = (t - mu) * lax.rsqrt(var + 1e-5) * gam_ref[:] + bet_ref[:]
  h = jnp.maximum(h, 0.0)
  o_ref[:] = jnp.dot(h, w1_ref[:], preferred_element_type=jnp.float32) * dis


def _tc_mid(s0, g0, dis_col, b0, gamma, beta, w1):
  return pl.pallas_call(
      _mid_body,
      grid=(_GRID,),
      in_specs=[
          pl.BlockSpec((NC, _BLK, D), lambda i: (0, i, 0)),
          pl.BlockSpec((_BLK, D), lambda i: (i, 0)),
          pl.BlockSpec((_BLK, 1), lambda i: (i, 0)),
          pl.BlockSpec((1, D), lambda i: (0, 0)),
          pl.BlockSpec((1, D), lambda i: (0, 0)),
          pl.BlockSpec((1, D), lambda i: (0, 0)),
          pl.BlockSpec((D, D), lambda i: (0, 0)),
      ],
      out_specs=pl.BlockSpec((_BLK, D), lambda i: (i, 0)),
      out_shape=jax.ShapeDtypeStruct((N_PAD, D), jnp.float32),
  )(s0, g0, dis_col, b0, gamma, beta, w1)


def _fin_body(s_ref, g_ref, dis_ref, b1_ref, o_ref):
  o_ref[:] = (s_ref[0] + s_ref[1] + g_ref[:]) * dis_ref[:] + b1_ref[:]


def _tc_fin(s1, g1, dis_col, b1):
  return pl.pallas_call(
      _fin_body,
      grid=(_GRID,),
      in_specs=[
          pl.BlockSpec((NC, _BLK, D), lambda i: (0, i, 0)),
          pl.BlockSpec((_BLK, D), lambda i: (i, 0)),
          pl.BlockSpec((_BLK, 1), lambda i: (i, 0)),
          pl.BlockSpec((1, D), lambda i: (0, 0)),
      ],
      out_specs=pl.BlockSpec((_BLK, D), lambda i: (i, 0)),
      out_shape=jax.ShapeDtypeStruct((N_PAD, D), jnp.float32),
  )(s1, g1, dis_col, b1)


def _split_edges(idx, e, ch0, ch1):
  ch_alloc = max(ch0, ch1) + KBLK
  e0 = min(e, NS * ch0 * EB)

  def dummies(k):
    return N_NODES + (jnp.arange(k, dtype=jnp.int32) % (N_PAD - N_NODES))

  w0 = jnp.concatenate([idx[:e0], dummies(NS * ch0 * EB - e0)]).reshape(
      NS, ch0, EB)
  w0 = jnp.concatenate(
      [w0, dummies(NS * (ch_alloc - ch0) * EB).reshape(NS, -1, EB)], axis=1)
  w1 = jnp.concatenate([idx[e0:],
                        dummies(NS * ch1 * EB - (e - e0))]).reshape(
      NS, ch1, EB)
  w1 = jnp.concatenate(
      [w1, dummies(NS * (ch_alloc - ch1) * EB).reshape(NS, -1, EB)], axis=1)
  return jnp.concatenate([w0, w1], axis=0)


def kernel(x, edge_index, W0, b0, gamma, beta, W1, b1):
  n, d = x.shape
  e = edge_index.shape[1]
  def ceil_to(a, m):
    return -(-a // m) * m

  t_chunks = -(-e // (NS * EB))
  ch0 = min(ceil_to(-(-t_chunks // 2), 2 * KBLK),
            ceil_to(t_chunks, 2 * KBLK))
  rem = max(e - NS * ch0 * EB, 0)
  ch1 = max(ceil_to(-(-rem // (NS * EB)), 2 * KBLK), 2 * KBLK)
  nblk2 = (ch0 // (2 * KBLK), ch1 // (2 * KBLK))

  src = edge_index[0].astype(jnp.int32)
  dst = edge_index[1].astype(jnp.int32)
  src_w = _split_edges(src, e, ch0, ch1)
  dst_w = _split_edges(dst, e, ch0, ch1)

  x_pad = jnp.zeros((N_PAD, d), x.dtype).at[:n].set(x)
  zeros_1d = jnp.zeros((ROWS_PER_TILE,), jnp.float32)
  zeros_2d = jnp.zeros((ROWS_PER_TILE, D), jnp.float32)
  ones_eb = jnp.ones((EB,), jnp.float32)
  b0r = b0.reshape(1, D)
  b1r = b1.reshape(1, D)
  gammar = gamma.reshape(1, D)
  betar = beta.reshape(1, D)

  deg_parts = _sc_degree(dst_w, zeros_1d, ones_eb, nblk2)
  dis2d = _tc_dis(deg_parts.reshape(NC, N_PAD // 128, 128))
  dis_col = dis2d.reshape(N_PAD, 1)

  g0 = _tc_g0(x_pad, W0, dis_col)
  s0 = _sc_scatter(g0, src_w, dst_w, zeros_2d, nblk2)
  g1 = _tc_mid(s0, g0, dis_col, b0r, gammar, betar, W1)
  s1 = _sc_scatter(g1, src_w, dst_w, zeros_2d, nblk2)
  out = _tc_fin(s1, g1, dis_col, b1r)
  return out[:n]


# speedup vs baseline: 3.5364x; 1.4186x over previous
import functools

import jax
import jax.numpy as jnp
from jax import lax
from jax.experimental import pallas as pl
from jax.experimental.pallas import tpu as pltpu
from jax.experimental.pallas import tpu_sc as plsc

N_NODES = 10000
D = 128
N_PAD = 10240
NC, NS = 2, 16
NW = NC * NS
EB = 128
KBLK = 4
ROWS_PER_TILE = N_PAD // NS


def _sc_degree(dst_w, zeros_1d, ones_eb, nblk2_by_core):
  @functools.partial(
      pl.kernel,
      out_type=jax.ShapeDtypeStruct((NC, N_PAD), jnp.float32),
      mesh=plsc.VectorSubcoreMesh(core_axis_name="c", subcore_axis_name="s"),
      scratch_types=[
          pltpu.VMEM((2, KBLK, EB), jnp.int32),
          pltpu.VMEM((EB,), jnp.float32),
          pltpu.VMEM_SHARED((N_PAD,), jnp.float32),
          pltpu.SemaphoreType.DMA,
      ],
  )
  def k(dst_hbm, z_hbm, ones_hbm, out_hbm, dst_v, ones_v, acc_sh, sem):
    c = lax.axis_index("c")
    s = lax.axis_index("s")
    wid = c * NS + s
    nblk2 = jnp.where(c == 0, nblk2_by_core[0], nblk2_by_core[1])
    pltpu.sync_copy(z_hbm, acc_sh.at[pl.ds(s * ROWS_PER_TILE, ROWS_PER_TILE)])
    pltpu.sync_copy(ones_hbm, ones_v)
    pltpu.async_copy(dst_hbm.at[wid].at[pl.ds(0, KBLK)], dst_v.at[0], sem)
    plsc.subcore_barrier()

    def blk(m, p):
      pltpu.make_async_copy(dst_hbm.at[wid].at[pl.ds(0, KBLK)], dst_v.at[p],
                            sem).wait()
      pltpu.async_copy(dst_hbm.at[wid].at[pl.ds((m + 1) * KBLK, KBLK)],
                       dst_v.at[1 - p], sem)
      for j in range(KBLK):
        pltpu.sync_copy(ones_v, acc_sh.at[dst_v.at[p].at[j]], add=True)

    def body(t, carry):
      blk(t * 2, 0)
      blk(t * 2 + 1, 1)
      return carry

    lax.fori_loop(0, nblk2, body, 0)
    pltpu.make_async_copy(dst_hbm.at[wid].at[pl.ds(0, KBLK)], dst_v.at[0],
                          sem).wait()
    plsc.subcore_barrier()
    pltpu.sync_copy(acc_sh.at[pl.ds(s * ROWS_PER_TILE, ROWS_PER_TILE)],
                    out_hbm.at[c].at[pl.ds(s * ROWS_PER_TILE, ROWS_PER_TILE)])

  return k(dst_w, zeros_1d, ones_eb)


def _sc_scatter(g, src_w, dst_w, zeros_2d, nblk2_by_core):
  @functools.partial(
      pl.kernel,
      out_type=jax.ShapeDtypeStruct((NC, N_PAD, D), jnp.float32),
      mesh=plsc.VectorSubcoreMesh(core_axis_name="c", subcore_axis_name="s"),
      scratch_types=[
          pltpu.VMEM((2, KBLK, EB), jnp.int32),
          pltpu.VMEM((2, KBLK, EB), jnp.int32),
          pltpu.VMEM((2, EB, D), jnp.float32),
          pltpu.VMEM_SHARED((N_PAD, D), jnp.float32),
          pltpu.SemaphoreType.DMA,
          pltpu.SemaphoreType.DMA,
          pltpu.SemaphoreType.DMA,
      ],
  )
  def k(g_hbm, src_hbm, dst_hbm, z_hbm, out_hbm, src_v, dst_v, rows_v, acc_sh,
        sem_i, sem_g0, sem_g1):
    c = lax.axis_index("c")
    s = lax.axis_index("s")
    wid = c * NS + s
    nblk2 = jnp.where(c == 0, nblk2_by_core[0], nblk2_by_core[1])
    sem_g = (sem_g0, sem_g1)

    def idx_load(m, p):
      pltpu.async_copy(src_hbm.at[wid].at[pl.ds(m * KBLK, KBLK)],
                       src_v.at[p], sem_i)
      pltpu.async_copy(dst_hbm.at[wid].at[pl.ds(m * KBLK, KBLK)],
                       dst_v.at[p], sem_i)

    def idx_wait(p):
      pltpu.make_async_copy(src_hbm.at[wid].at[pl.ds(0, KBLK)], src_v.at[p],
                            sem_i).wait()
      pltpu.make_async_copy(dst_hbm.at[wid].at[pl.ds(0, KBLK)], dst_v.at[p],
                            sem_i).wait()

    def gather(p, j, b):
      pltpu.async_copy(g_hbm.at[src_v.at[p].at[j]], rows_v.at[b], sem_g[b])

    def gather_wait(b):
      pltpu.make_async_copy(g_hbm.at[src_v.at[0].at[0]], rows_v.at[b],
                            sem_g[b]).wait()

    with jax.named_scope("zfill"):
      pltpu.sync_copy(z_hbm, acc_sh.at[pl.ds(s * ROWS_PER_TILE, ROWS_PER_TILE)])
      idx_load(0, 0)
      plsc.subcore_barrier()
      idx_wait(0)
      idx_load(1, 1)
      gather(0, 0, 0)
      gather(0, 1, 1)

    def blk(m, p):
      for j in range(KBLK):
        b = j % 2
        gather_wait(b)
        pltpu.sync_copy(rows_v.at[b], acc_sh.at[dst_v.at[p].at[j]], add=True)
        if j == KBLK - 2:
          idx_wait(1 - p)
        if j < KBLK - 2:
          gather(p, j + 2, b)
        else:
          gather(1 - p, j + 2 - KBLK, b)
        if j == KBLK - 1:
          idx_load(m + 2, p)

    def body(t, carry):
      blk(t * 2, 0)
      blk(t * 2 + 1, 1)
      return carry

    with jax.named_scope("eloop"):
      lax.fori_loop(0, nblk2, body, 0)
      gather_wait(0)
      gather_wait(1)
      idx_wait(0)
    with jax.named_scope("obar"):
      plsc.subcore_barrier()
    with jax.named_scope("oout"):
      pltpu.sync_copy(acc_sh.at[pl.ds(s * ROWS_PER_TILE, ROWS_PER_TILE)],
                      out_hbm.at[c].at[pl.ds(s * ROWS_PER_TILE, ROWS_PER_TILE)])

  return k(g, src_w, dst_w, zeros_2d)


_BLK = 1024
_GRID = N_PAD // _BLK


def _dis_body(p_ref, o_ref):
  o_ref[:] = lax.rsqrt(p_ref[0] + p_ref[1] + 1.0)


def _tc_dis(deg_parts):
  return pl.pallas_call(
      _dis_body,
      out_shape=jax.ShapeDtypeStruct((N_PAD // 128, 128), jnp.float32),
  )(deg_parts)


def _g0_body(x_ref, w_ref, dis_ref, o_ref):
  m = jnp.dot(x_ref[:], w_ref[:], preferred_element_type=jnp.float32)
  o_ref[:] = m * dis_ref[:]


def _tc_g0(x_pad, w0, dis_col):
  return pl.pallas_call(
      _g0_body,
      grid=(_GRID,),
      in_specs=[
          pl.BlockSpec((_BLK, D), lambda i: (i, 0)),
          pl.BlockSpec((D, D), lambda i: (0, 0)),
          pl.BlockSpec((_BLK, 1), lambda i: (i, 0)),
      ],
      out_specs=pl.BlockSpec((_BLK, D), lambda i: (i, 0)),
      out_shape=jax.ShapeDtypeStruct((N_PAD, D), jnp.float32),
  )(x_pad, w0, dis_col)


def _mid_body(s_ref, g_ref, dis_ref, b0_ref, gam_ref, bet_ref, w1_ref, o_ref):
  dis = dis_ref[:]
  t = (s_ref[0] + s_ref[1] + g_ref[:]) * dis + b0_ref[:]
  mu = jnp.mean(t, axis=1, keepdims=True)
  var = jnp.mean((t - mu) * (t - mu), axis=1, keepdims=True)
  h = (t - mu) * lax.rsqrt(var + 1e-5) * gam_ref[:] + bet_ref[:]
  h = jnp.maximum(h, 0.0)
  o_ref[:] = jnp.dot(h, w1_ref[:], preferred_element_type=jnp.float32) * dis


def _tc_mid(s0, g0, dis_col, b0, gamma, beta, w1):
  return pl.pallas_call(
      _mid_body,
      grid=(_GRID,),
      in_specs=[
          pl.BlockSpec((NC, _BLK, D), lambda i: (0, i, 0)),
          pl.BlockSpec((_BLK, D), lambda i: (i, 0)),
          pl.BlockSpec((_BLK, 1), lambda i: (i, 0)),
          pl.BlockSpec((1, D), lambda i: (0, 0)),
          pl.BlockSpec((1, D), lambda i: (0, 0)),
          pl.BlockSpec((1, D), lambda i: (0, 0)),
          pl.BlockSpec((D, D), lambda i: (0, 0)),
      ],
      out_specs=pl.BlockSpec((_BLK, D), lambda i: (i, 0)),
      out_shape=jax.ShapeDtypeStruct((N_PAD, D), jnp.float32),
  )(s0, g0, dis_col, b0, gamma, beta, w1)


def _fin_body(s_ref, g_ref, dis_ref, b1_ref, o_ref):
  o_ref[:] = (s_ref[0] + s_ref[1] + g_ref[:]) * dis_ref[:] + b1_ref[:]


def _tc_fin(s1, g1, dis_col, b1):
  return pl.pallas_call(
      _fin_body,
      grid=(_GRID,),
      in_specs=[
          pl.BlockSpec((NC, _BLK, D), lambda i: (0, i, 0)),
          pl.BlockSpec((_BLK, D), lambda i: (i, 0)),
          pl.BlockSpec((_BLK, 1), lambda i: (i, 0)),
          pl.BlockSpec((1, D), lambda i: (0, 0)),
      ],
      out_specs=pl.BlockSpec((_BLK, D), lambda i: (i, 0)),
      out_shape=jax.ShapeDtypeStruct((N_PAD, D), jnp.float32),
  )(s1, g1, dis_col, b1)


def _split_edges(idx, e, ch0, ch1):
  ch_alloc = max(ch0, ch1) + 2 * KBLK
  e0 = min(e, NS * ch0 * EB)

  def dummies(k):
    return N_NODES + (jnp.arange(k, dtype=jnp.int32) % (N_PAD - N_NODES))

  w0 = jnp.concatenate([idx[:e0], dummies(NS * ch0 * EB - e0)]).reshape(
      NS, ch0, EB)
  w0 = jnp.concatenate(
      [w0, dummies(NS * (ch_alloc - ch0) * EB).reshape(NS, -1, EB)], axis=1)
  w1 = jnp.concatenate([idx[e0:],
                        dummies(NS * ch1 * EB - (e - e0))]).reshape(
      NS, ch1, EB)
  w1 = jnp.concatenate(
      [w1, dummies(NS * (ch_alloc - ch1) * EB).reshape(NS, -1, EB)], axis=1)
  return jnp.concatenate([w0, w1], axis=0)


def kernel(x, edge_index, W0, b0, gamma, beta, W1, b1):
  n, d = x.shape
  e = edge_index.shape[1]
  def ceil_to(a, m):
    return -(-a // m) * m

  t_chunks = -(-e // (NS * EB))
  ch0 = min(ceil_to(-(-t_chunks // 2), 2 * KBLK),
            ceil_to(t_chunks, 2 * KBLK))
  rem = max(e - NS * ch0 * EB, 0)
  ch1 = max(ceil_to(-(-rem // (NS * EB)), 2 * KBLK), 2 * KBLK)
  nblk2 = (ch0 // (2 * KBLK), ch1 // (2 * KBLK))

  src = edge_index[0].astype(jnp.int32)
  dst = edge_index[1].astype(jnp.int32)
  src_w = _split_edges(src, e, ch0, ch1)
  dst_w = _split_edges(dst, e, ch0, ch1)

  x_pad = jnp.zeros((N_PAD, d), x.dtype).at[:n].set(x)
  zeros_1d = jnp.zeros((ROWS_PER_TILE,), jnp.float32)
  zeros_2d = jnp.zeros((ROWS_PER_TILE, D), jnp.float32)
  ones_eb = jnp.ones((EB,), jnp.float32)
  b0r = b0.reshape(1, D)
  b1r = b1.reshape(1, D)
  gammar = gamma.reshape(1, D)
  betar = beta.reshape(1, D)

  deg_parts = _sc_degree(dst_w, zeros_1d, ones_eb, nblk2)
  dis2d = _tc_dis(deg_parts.reshape(NC, N_PAD // 128, 128))
  dis_col = dis2d.reshape(N_PAD, 1)

  g0 = _tc_g0(x_pad, W0, dis_col)
  s0 = _sc_scatter(g0, src_w, dst_w, zeros_2d, nblk2)
  g1 = _tc_mid(s0, g0, dis_col, b0r, gammar, betar, W1)
  s1 = _sc_scatter(g1, src_w, dst_w, zeros_2d, nblk2)
  out = _tc_fin(s1, g1, dis_col, b1r)
  return out[:n]
